# baseline (speedup 1.0000x reference)
import functools

import jax
import jax.numpy as jnp
from jax import lax
from jax.experimental import pallas as pl

NUM_BINS = 128
HEAD_DIM = 128
NUM_FREQS = HEAD_DIM // 2
NUM_QUERIES = 4096
NUM_KEYS = 32768
EPS = 1e-08

GRID = 2
KBLK = NUM_KEYS // GRID
QBLK = NUM_QUERIES // GRID


def _fused_kernel(k_ref, probes_ref, probes_sw_ref, cos_i_ref, sin_i_ref,
                  qxt_ref, qyt_ref, pxq_ref, pyq_ref, cosf_ref, sinf_ref,
                  wraw_ref, bias_ref, kout_ref, qout_ref):
    rot_p = probes_ref[...] * cos_i_ref[...] + probes_sw_ref[...] * sin_i_ref[...]
    kout_ref[...] = lax.dot_general(
        k_ref[...], rot_p,
        dimension_numbers=(((1,), (1,)), ((), ())),
        preferred_element_type=jnp.float32,
        precision=lax.Precision.DEFAULT)

    cosf = cosf_ref[...]
    sinf = sinf_ref[...]
    pxq = pxq_ref[...]
    pyq = pyq_ref[...]
    px = pxq * cosf - pyq * sinf
    py = pxq * sinf + pyq * cosf

    wraw = wraw_ref[...]
    w = -(jnp.maximum(wraw, 0.0) + jnp.log1p(jnp.exp(-jnp.abs(wraw))))

    qxt = qxt_ref[...]
    qyt = qyt_ref[...]
    sq = qxt * qxt + qyt * qyt
    tp = px * px + py * py + EPS
    w2 = w * w
    ones_q = jnp.ones((NUM_FREQS, 1, QBLK), dtype=jnp.float32)
    lhs = jnp.concatenate(
        [qxt[:, None, :], qyt[:, None, :], sq[:, None, :], ones_q], axis=1)
    rhs = jnp.concatenate(
        [(-2.0 * px * w2)[:, None, :], (-2.0 * py * w2)[:, None, :],
         w2[:, None, :], (tp * w2)[:, None, :]], axis=1)
    FCHUNK = 16
    acc = jnp.zeros((QBLK, NUM_BINS), dtype=jnp.float32)
    for c in range(0, NUM_FREQS, FCHUNK):
        s2 = lax.dot_general(
            lhs[c:c + FCHUNK], rhs[c:c + FCHUNK],
            dimension_numbers=(((1,), (1,)), ((0,), (0,))),
            preferred_element_type=jnp.float32,
            precision=lax.Precision.DEFAULT)
        s2 = jnp.maximum(s2, 1e-35)
        acc = acc + jnp.sum(s2 * lax.rsqrt(s2), axis=0)
    qout_ref[...] = bias_ref[...] - acc


@functools.partial(jax.jit, static_argnums=())
def kernel(Q, K, reference_angles, probes, q_weights_raw, q_bias):
    cos_f = jnp.cos(reference_angles)
    sin_f = jnp.sin(reference_angles)
    cos_i = jnp.repeat(cos_f, 2).reshape(1, HEAD_DIM)
    sin_i = jnp.stack([-sin_f, sin_f], axis=-1).reshape(1, HEAD_DIM)
    probes_sw = probes.reshape(NUM_BINS, NUM_FREQS, 2)[..., ::-1].reshape(
        NUM_BINS, HEAD_DIM)

    qf = Q.reshape(NUM_QUERIES, NUM_FREQS, 2)
    qxt = qf[..., 0].T
    qyt = qf[..., 1].T
    pf = probes.reshape(NUM_BINS, NUM_FREQS, 2)
    pxq = pf[..., 0].T
    pyq = pf[..., 1].T
    cosf_col = cos_f.reshape(NUM_FREQS, 1)
    sinf_col = sin_f.reshape(NUM_FREQS, 1)
    wraw_t = q_weights_raw.T
    bias_row = q_bias.reshape(1, NUM_BINS)

    full = lambda shape: pl.BlockSpec(shape, lambda i: (0, 0))
    kout, qout = pl.pallas_call(
        _fused_kernel,
        grid=(GRID,),
        in_specs=[
            pl.BlockSpec((KBLK, HEAD_DIM), lambda i: (i, 0)),
            full((NUM_BINS, HEAD_DIM)),
            full((NUM_BINS, HEAD_DIM)),
            full((1, HEAD_DIM)),
            full((1, HEAD_DIM)),
            pl.BlockSpec((NUM_FREQS, QBLK), lambda i: (0, i)),
            pl.BlockSpec((NUM_FREQS, QBLK), lambda i: (0, i)),
            full((NUM_FREQS, NUM_BINS)),
            full((NUM_FREQS, NUM_BINS)),
            full((NUM_FREQS, 1)),
            full((NUM_FREQS, 1)),
            full((NUM_FREQS, NUM_BINS)),
            full((1, NUM_BINS)),
        ],
        out_specs=[
            pl.BlockSpec((KBLK, NUM_BINS), lambda i: (i, 0)),
            pl.BlockSpec((QBLK, NUM_BINS), lambda i: (i, 0)),
        ],
        out_shape=[
            jax.ShapeDtypeStruct((NUM_KEYS, NUM_BINS), jnp.float32),
            jax.ShapeDtypeStruct((NUM_QUERIES, NUM_BINS), jnp.float32),
        ],
    )(K, probes, probes_sw, cos_i, sin_i, qxt, qyt, pxq, pyq,
      cosf_col, sinf_col, wraw_t, bias_row)
    return (qout, kout)


# speedup vs baseline: 1.0281x; 1.0281x over previous
import functools

import jax
import jax.numpy as jnp
from jax import lax
from jax.experimental import pallas as pl

NUM_BINS = 128
HEAD_DIM = 128
NUM_FREQS = HEAD_DIM // 2
NUM_QUERIES = 4096
NUM_KEYS = 32768
EPS = 1e-08

GRID = 4
KBLK = NUM_KEYS // GRID
QBLK = NUM_QUERIES // GRID


def _fused_kernel(k_ref, probes_ref, probes_sw_ref, cos_i_ref, sin_i_ref,
                  qxt_ref, qyt_ref, pxq_ref, pyq_ref, cosf_ref, sinf_ref,
                  wraw_ref, bias_ref, kout_ref, qout_ref):
    rot_p = probes_ref[...] * cos_i_ref[...] + probes_sw_ref[...] * sin_i_ref[...]
    kout_ref[...] = lax.dot_general(
        k_ref[...], rot_p,
        dimension_numbers=(((1,), (1,)), ((), ())),
        preferred_element_type=jnp.float32,
        precision=lax.Precision.DEFAULT)

    cosf = cosf_ref[...]
    sinf = sinf_ref[...]
    pxq = pxq_ref[...]
    pyq = pyq_ref[...]
    px = pxq * cosf - pyq * sinf
    py = pxq * sinf + pyq * cosf

    wraw = wraw_ref[...]
    w = -(jnp.maximum(wraw, 0.0) + jnp.log1p(jnp.exp(-jnp.abs(wraw))))

    qxt = qxt_ref[...]
    qyt = qyt_ref[...]
    sq = qxt * qxt + qyt * qyt
    tp = px * px + py * py + EPS
    w2 = w * w
    ones_q = jnp.ones((NUM_FREQS, 1, QBLK), dtype=jnp.float32)
    lhs = jnp.concatenate(
        [qxt[:, None, :], qyt[:, None, :], sq[:, None, :], ones_q], axis=1)
    rhs = jnp.concatenate(
        [(-2.0 * px * w2)[:, None, :], (-2.0 * py * w2)[:, None, :],
         w2[:, None, :], (tp * w2)[:, None, :]], axis=1)
    s2 = lax.dot_general(
        lhs, rhs,
        dimension_numbers=(((1,), (1,)), ((0,), (0,))),
        preferred_element_type=jnp.float32,
        precision=lax.Precision.DEFAULT)
    s2 = jnp.maximum(s2, 1e-35)
    qout_ref[...] = bias_ref[...] - jnp.sum(s2 * lax.rsqrt(s2), axis=0)


@functools.partial(jax.jit, static_argnums=())
def kernel(Q, K, reference_angles, probes, q_weights_raw, q_bias):
    cos_f = jnp.cos(reference_angles)
    sin_f = jnp.sin(reference_angles)
    cos_i = jnp.repeat(cos_f, 2).reshape(1, HEAD_DIM)
    sin_i = jnp.stack([-sin_f, sin_f], axis=-1).reshape(1, HEAD_DIM)
    probes_sw = probes.reshape(NUM_BINS, NUM_FREQS, 2)[..., ::-1].reshape(
        NUM_BINS, HEAD_DIM)

    qf = Q.reshape(NUM_QUERIES, NUM_FREQS, 2)
    qxt = qf[..., 0].T
    qyt = qf[..., 1].T
    pf = probes.reshape(NUM_BINS, NUM_FREQS, 2)
    pxq = pf[..., 0].T
    pyq = pf[..., 1].T
    cosf_col = cos_f.reshape(NUM_FREQS, 1)
    sinf_col = sin_f.reshape(NUM_FREQS, 1)
    wraw_t = q_weights_raw.T
    bias_row = q_bias.reshape(1, NUM_BINS)

    full = lambda shape: pl.BlockSpec(shape, lambda i: (0, 0))
    kout, qout = pl.pallas_call(
        _fused_kernel,
        grid=(GRID,),
        in_specs=[
            pl.BlockSpec((KBLK, HEAD_DIM), lambda i: (i, 0)),
            full((NUM_BINS, HEAD_DIM)),
            full((NUM_BINS, HEAD_DIM)),
            full((1, HEAD_DIM)),
            full((1, HEAD_DIM)),
            pl.BlockSpec((NUM_FREQS, QBLK), lambda i: (0, i)),
            pl.BlockSpec((NUM_FREQS, QBLK), lambda i: (0, i)),
            full((NUM_FREQS, NUM_BINS)),
            full((NUM_FREQS, NUM_BINS)),
            full((NUM_FREQS, 1)),
            full((NUM_FREQS, 1)),
            full((NUM_FREQS, NUM_BINS)),
            full((1, NUM_BINS)),
        ],
        out_specs=[
            pl.BlockSpec((KBLK, NUM_BINS), lambda i: (i, 0)),
            pl.BlockSpec((QBLK, NUM_BINS), lambda i: (i, 0)),
        ],
        out_shape=[
            jax.ShapeDtypeStruct((NUM_KEYS, NUM_BINS), jnp.float32),
            jax.ShapeDtypeStruct((NUM_QUERIES, NUM_BINS), jnp.float32),
        ],
    )(K, probes, probes_sw, cos_i, sin_i, qxt, qyt, pxq, pyq,
      cosf_col, sinf_col, wraw_t, bias_row)
    return (qout, kout)
